# baseline (speedup 1.0000x reference)
import jax
import jax.numpy as jnp
from jax.experimental import pallas as pl
from jax.experimental.pallas import tpu as pltpu

D_MODEL = 1024
FPB = 43
BPB = 4
BPP = 4
DPS = D_MODEL // 4
BS = 256


def _add_pe_kernel(fe_ref, be_ref, ba_ref, x_ref, pe_ref, o_ref):
    j = pl.program_id(0)
    row = j * BS + jax.lax.broadcasted_iota(jnp.int32, (BS, 1), 0)
    beat_pos = row % FPB
    bar_pos = (row // FPB) % BPB
    phrase_pos = (row // (FPB * BPB)) % BPP
    cols48 = jax.lax.broadcasted_iota(jnp.int32, (BS, 48), 1)
    cols8 = jax.lax.broadcasted_iota(jnp.int32, (BS, 8), 1)
    oh_f = (cols48 == beat_pos).astype(jnp.float32)
    oh_b = (cols8 == bar_pos).astype(jnp.float32)
    oh_p = (cols8 == phrase_pos).astype(jnp.float32)
    f = jnp.dot(oh_f, fe_ref[...], preferred_element_type=jnp.float32)
    b = jnp.dot(oh_b, be_ref[...], preferred_element_type=jnp.float32)
    p = jnp.dot(oh_p, ba_ref[...], preferred_element_type=jnp.float32)
    enc = jnp.concatenate([f, b, p, pe_ref[...]], axis=-1)
    o_ref[...] = x_ref[...] + enc[None, :, :]


def kernel(x, frame_embed, beat_embed, bar_embed, pe):
    B, S, D = x.shape
    fe = jnp.zeros((48, DPS), x.dtype).at[:FPB].set(frame_embed)
    be = jnp.zeros((8, DPS), x.dtype).at[:BPB].set(beat_embed)
    ba = jnp.zeros((8, DPS), x.dtype).at[:BPP].set(bar_embed)
    return pl.pallas_call(
        _add_pe_kernel,
        grid=(S // BS,),
        in_specs=[
            pl.BlockSpec((48, DPS), lambda j: (0, 0)),
            pl.BlockSpec((8, DPS), lambda j: (0, 0)),
            pl.BlockSpec((8, DPS), lambda j: (0, 0)),
            pl.BlockSpec((B, BS, D), lambda j: (0, j, 0)),
            pl.BlockSpec((BS, DPS), lambda j: (j, 0)),
        ],
        out_specs=pl.BlockSpec((B, BS, D), lambda j: (0, j, 0)),
        out_shape=jax.ShapeDtypeStruct((B, S, D), x.dtype),
        compiler_params=pltpu.CompilerParams(
            dimension_semantics=("parallel",),
        ),
    )(fe, be, ba, x, pe)


# speedup vs baseline: 1.0118x; 1.0118x over previous
import math

import jax
import jax.numpy as jnp
from jax.experimental import pallas as pl
from jax.experimental.pallas import tpu as pltpu

D_MODEL = 1024
FPB = 43
BPB = 4
BPP = 4
DPS = D_MODEL // 4
BS = 512


def _add_pe_kernel(fe_ref, be_ref, ba_ref, fp_ref, x_ref, o_ref):
    j = pl.program_id(0)
    row = j * BS + jax.lax.broadcasted_iota(jnp.int32, (BS, 1), 0)
    beat_pos = row % FPB
    bar_pos = (row // FPB) % BPB
    phrase_pos = (row // (FPB * BPB)) % BPP
    cols48 = jax.lax.broadcasted_iota(jnp.int32, (BS, 48), 1)
    cols8 = jax.lax.broadcasted_iota(jnp.int32, (BS, 8), 1)
    oh_f = (cols48 == beat_pos).astype(jnp.float32)
    oh_b = (cols8 == bar_pos).astype(jnp.float32)
    oh_p = (cols8 == phrase_pos).astype(jnp.float32)
    f = jnp.dot(oh_f, fe_ref[...], preferred_element_type=jnp.float32)
    b = jnp.dot(oh_b, be_ref[...], preferred_element_type=jnp.float32)
    p = jnp.dot(oh_p, ba_ref[...], preferred_element_type=jnp.float32)
    freq = fp_ref[0:1, :]
    phase = fp_ref[1:2, :]
    abs_pe = jnp.sin(row.astype(jnp.float32) * freq + phase)
    enc = jnp.concatenate([f, b, p, abs_pe], axis=-1)
    o_ref[...] = x_ref[...] + enc[None, :, :]


def kernel(x, frame_embed, beat_embed, bar_embed, pe):
    B, S, D = x.shape
    fe = jnp.zeros((48, DPS), x.dtype).at[:FPB].set(frame_embed)
    be = jnp.zeros((8, DPS), x.dtype).at[:BPB].set(beat_embed)
    ba = jnp.zeros((8, DPS), x.dtype).at[:BPP].set(bar_embed)
    lane = jnp.arange(DPS)
    freq = jnp.exp((lane // 2 * 2).astype(jnp.float32) * (-math.log(10000.0) / DPS))
    phase = jnp.where(lane % 2 == 1, jnp.float32(math.pi / 2), jnp.float32(0.0))
    fp = jnp.zeros((8, DPS), x.dtype).at[0].set(freq).at[1].set(phase)
    return pl.pallas_call(
        _add_pe_kernel,
        grid=(S // BS,),
        in_specs=[
            pl.BlockSpec((48, DPS), lambda j: (0, 0)),
            pl.BlockSpec((8, DPS), lambda j: (0, 0)),
            pl.BlockSpec((8, DPS), lambda j: (0, 0)),
            pl.BlockSpec((8, DPS), lambda j: (0, 0)),
            pl.BlockSpec((B, BS, D), lambda j: (0, j, 0)),
        ],
        out_specs=pl.BlockSpec((B, BS, D), lambda j: (0, j, 0)),
        out_shape=jax.ShapeDtypeStruct((B, S, D), x.dtype),
        compiler_params=pltpu.CompilerParams(
            dimension_semantics=("parallel",),
        ),
    )(fe, be, ba, fp, x)


# speedup vs baseline: 1.0540x; 1.0417x over previous
import math

import jax
import jax.numpy as jnp
from jax.experimental import pallas as pl
from jax.experimental.pallas import tpu as pltpu

D_MODEL = 1024
FPB = 43
BPB = 4
BPP = 4
DPS = D_MODEL // 4
BS = 512


def _add_pe_kernel(fe_ref, be_ref, ba_ref, fp_ref, x_ref, o_ref):
    j = pl.program_id(0)
    row = j * BS + jax.lax.broadcasted_iota(jnp.int32, (BS, 1), 0)
    beat_pos = row % FPB
    bar_pos = (row // FPB) % BPB
    phrase_pos = (row // (FPB * BPB)) % BPP
    cols43 = jax.lax.broadcasted_iota(jnp.int32, (BS, FPB), 1)
    cols4 = jax.lax.broadcasted_iota(jnp.int32, (BS, BPB), 1)
    oh_f = (cols43 == beat_pos).astype(jnp.float32)
    oh_b = (cols4 == bar_pos).astype(jnp.float32)
    oh_p = (cols4 == phrase_pos).astype(jnp.float32)
    f = jnp.dot(oh_f, fe_ref[...], preferred_element_type=jnp.float32)
    b = jnp.dot(oh_b, be_ref[...], preferred_element_type=jnp.float32)
    p = jnp.dot(oh_p, ba_ref[...], preferred_element_type=jnp.float32)
    freq = fp_ref[0:1, :]
    phase = fp_ref[1:2, :]
    abs_pe = jnp.sin(row.astype(jnp.float32) * freq + phase)
    enc = jnp.concatenate([f, b, p, abs_pe], axis=-1)
    o_ref[...] = x_ref[...] + enc[None, :, :]


def kernel(x, frame_embed, beat_embed, bar_embed, pe):
    B, S, D = x.shape
    lane = jnp.arange(DPS)
    freq = jnp.exp((lane // 2 * 2).astype(jnp.float32) * (-math.log(10000.0) / DPS))
    phase = jnp.where(lane % 2 == 1, jnp.float32(math.pi / 2), jnp.float32(0.0))
    fp = jnp.zeros((8, DPS), x.dtype).at[0].set(freq).at[1].set(phase)
    return pl.pallas_call(
        _add_pe_kernel,
        grid=(S // BS,),
        in_specs=[
            pl.BlockSpec((FPB, DPS), lambda j: (0, 0)),
            pl.BlockSpec((BPB, DPS), lambda j: (0, 0)),
            pl.BlockSpec((BPP, DPS), lambda j: (0, 0)),
            pl.BlockSpec((8, DPS), lambda j: (0, 0)),
            pl.BlockSpec((B, BS, D), lambda j: (0, j, 0)),
        ],
        out_specs=pl.BlockSpec((B, BS, D), lambda j: (0, j, 0)),
        out_shape=jax.ShapeDtypeStruct((B, S, D), x.dtype),
        compiler_params=pltpu.CompilerParams(
            dimension_semantics=("parallel",),
        ),
    )(frame_embed, beat_embed, bar_embed, fp, x)
